# baseline (speedup 1.0000x reference)
import jax
import jax.numpy as jnp
from jax.experimental import pallas as pl
from jax.experimental.pallas import tpu as pltpu

_TOKENS = 16384
_PATHS = 64
_CHUNK = 2048
_NCHUNK = _TOKENS // _CHUNK


def _body(s_hbm, o_hbm, s_v, o_v, sem_in, sem_out):
    def start_in(k, buf):
        pltpu.make_async_copy(
            s_hbm.at[pl.ds(k * _CHUNK, _CHUNK)], s_v.at[buf], sem_in.at[buf]
        ).start()

    def wait_in(buf):
        pltpu.make_async_copy(
            s_hbm.at[pl.ds(0, _CHUNK)], s_v.at[buf], sem_in.at[buf]
        ).wait()

    def start_out(k, buf):
        pltpu.make_async_copy(
            o_v.at[buf], o_hbm.at[pl.ds(k * _CHUNK, _CHUNK)], sem_out.at[buf]
        ).start()

    def wait_out(buf):
        pltpu.make_async_copy(
            o_v.at[buf], o_hbm.at[pl.ds(0, _CHUNK)], sem_out.at[buf]
        ).wait()

    start_in(0, 0)
    for k in range(_NCHUNK):
        buf = k % 2
        if k + 1 < _NCHUNK:
            start_in(k + 1, (k + 1) % 2)
        wait_in(buf)
        if k >= 2:
            wait_out(buf)
        s = s_v[buf]
        pos = s > 0.0
        col = jax.lax.broadcasted_iota(jnp.int32, s.shape, 1)
        rmax = jnp.max(s, axis=1, keepdims=True)
        resid = (col == 0) & (rmax <= 0.0)
        o_v[buf] = jnp.where(pos | resid, 1, 0).astype(jnp.int32)
        start_out(k, buf)
    wait_out((_NCHUNK - 2) % 2)
    wait_out((_NCHUNK - 1) % 2)


def kernel(score):
    return pl.pallas_call(
        _body,
        out_shape=jax.ShapeDtypeStruct((_TOKENS, _PATHS), jnp.int32),
        in_specs=[pl.BlockSpec(memory_space=pl.ANY)],
        out_specs=pl.BlockSpec(memory_space=pl.ANY),
        scratch_shapes=[
            pltpu.VMEM((2, _CHUNK, _PATHS), jnp.float32),
            pltpu.VMEM((2, _CHUNK, _PATHS), jnp.int32),
            pltpu.SemaphoreType.DMA((2,)),
            pltpu.SemaphoreType.DMA((2,)),
        ],
    )(score)


# speedup vs baseline: 1.0003x; 1.0003x over previous
import jax
import jax.numpy as jnp
from jax.experimental import pallas as pl
from jax.experimental.pallas import tpu as pltpu

_TOKENS = 16384
_PATHS = 64
_CHUNK = 2048
_NCHUNK = _TOKENS // _CHUNK


def _body(s_hbm, o_hbm, s_v, o_v, sem_in, sem_out):
    def start_in(k, buf):
        pltpu.make_async_copy(
            s_hbm.at[pl.ds(k * _CHUNK, _CHUNK)], s_v.at[buf], sem_in.at[buf]
        ).start()

    def wait_in(buf):
        pltpu.make_async_copy(
            s_hbm.at[pl.ds(0, _CHUNK)], s_v.at[buf], sem_in.at[buf]
        ).wait()

    def start_out(k, buf):
        pltpu.async_copy(
            o_v.at[buf], o_hbm.at[pl.ds(k * _CHUNK, _CHUNK)], sem_out.at[buf],
            priority=1,
        )

    def wait_out(buf):
        pltpu.make_async_copy(
            o_v.at[buf], o_hbm.at[pl.ds(0, _CHUNK)], sem_out.at[buf]
        ).wait()

    start_in(0, 0)
    for k in range(_NCHUNK):
        buf = k % 2
        if k + 1 < _NCHUNK:
            start_in(k + 1, (k + 1) % 2)
        wait_in(buf)
        if k >= 2:
            wait_out(buf)
        s = s_v[buf]
        pos = s > 0.0
        col = jax.lax.broadcasted_iota(jnp.int32, s.shape, 1)
        rmax = jnp.max(s, axis=1, keepdims=True)
        resid = (col == 0) & (rmax <= 0.0)
        o_v[buf] = jnp.where(pos | resid, 1, 0).astype(jnp.int32)
        start_out(k, buf)
    wait_out((_NCHUNK - 2) % 2)
    wait_out((_NCHUNK - 1) % 2)


def kernel(score):
    return pl.pallas_call(
        _body,
        out_shape=jax.ShapeDtypeStruct((_TOKENS, _PATHS), jnp.int32),
        in_specs=[pl.BlockSpec(memory_space=pl.ANY)],
        out_specs=pl.BlockSpec(memory_space=pl.ANY),
        scratch_shapes=[
            pltpu.VMEM((2, _CHUNK, _PATHS), jnp.float32),
            pltpu.VMEM((2, _CHUNK, _PATHS), jnp.int32),
            pltpu.SemaphoreType.DMA((2,)),
            pltpu.SemaphoreType.DMA((2,)),
        ],
    )(score)


# speedup vs baseline: 1.1295x; 1.1292x over previous
import jax
import jax.numpy as jnp
from jax.experimental import pallas as pl

_TOKENS = 16384
_PATHS = 64
_BLOCK_ROWS = 8192


def _body(s_ref, o_ref):
    s = s_ref[...]
    pos = s > 0.0
    col = jax.lax.broadcasted_iota(jnp.int32, s.shape, 1)
    rmax = jnp.max(s, axis=1, keepdims=True)
    resid = (col == 0) & (rmax <= 0.0)
    o_ref[...] = jnp.where(pos | resid, 1, 0).astype(jnp.int32)


def kernel(score):
    return pl.pallas_call(
        _body,
        out_shape=jax.ShapeDtypeStruct((_TOKENS, _PATHS), jnp.int32),
        grid=(_TOKENS // _BLOCK_ROWS,),
        in_specs=[pl.BlockSpec((_BLOCK_ROWS, _PATHS), lambda i: (i, 0))],
        out_specs=pl.BlockSpec((_BLOCK_ROWS, _PATHS), lambda i: (i, 0)),
    )(score)
